# Initial kernel scaffold; baseline (speedup 1.0000x reference)
import jax
import jax.numpy as jnp
from jax.experimental import pallas as pl


def kernel(z_ins, bag_idx, bag_instances, instance_mu, instance_std, W, b):
    raise NotImplementedError("write your pallas kernel here")


# speedup vs baseline: 10.0684x; 10.0684x over previous
import jax
import jax.numpy as jnp
from jax.experimental import pallas as pl
from jax.experimental.pallas import tpu as pltpu

_N = 320000
_D = 128
_B = 10000
_BN = 3200


def _matvec_body(x_ref, w_ref, b_ref, o_ref):
    i = pl.program_id(0)
    o_ref[pl.ds(pl.multiple_of(i * _BN, 128), _BN)] = (
        jnp.dot(x_ref[...].astype(jnp.bfloat16), w_ref[...].astype(jnp.bfloat16).T,
                preferred_element_type=jnp.float32)[:, 0]
        + b_ref[0, 0])


def _matvec(z_ins, W, b):
    n = z_ins.shape[0]
    grid = n // _BN
    return pl.pallas_call(
        _matvec_body,
        grid=(grid,),
        in_specs=[
            pl.BlockSpec((_BN, _D), lambda i: (i, 0)),
            pl.BlockSpec((1, _D), lambda i: (0, 0)),
            pl.BlockSpec((1, 1), lambda i: (0, 0)),
        ],
        out_specs=pl.BlockSpec((n,), lambda i: (0,)),
        out_shape=jax.ShapeDtypeStruct((n,), jnp.float32),
    )(z_ins, W, b.reshape(1, 1))


def kernel(z_ins, bag_idx, bag_instances, instance_mu, instance_std, W, b):
    n = z_ins.shape[0]
    score = _matvec(z_ins, W, b)
    loc_ins = score[:, None]
    seg = bag_idx
    M = jax.ops.segment_max(score, seg, num_segments=_B, indices_are_sorted=True)
    pos = jnp.arange(n, dtype=jnp.int32)
    cand = jnp.where(score == M[seg], pos, n)
    gmax = jax.ops.segment_min(cand, seg, num_segments=_B, indices_are_sorted=True)
    starts = jax.ops.segment_min(pos, seg, num_segments=_B, indices_are_sorted=True)
    local = gmax - starts
    max_z_ins = z_ins[local]
    max_instances = bag_instances[local]
    max_instances_mu = instance_mu[local]
    max_instances_std = instance_std[local]
    return (M[:, None], max_instances, max_z_ins, loc_ins,
            max_instances_mu, max_instances_std)


# speedup vs baseline: 79.5453x; 7.9005x over previous
import functools

import jax
import jax.numpy as jnp
from jax import lax
from jax.experimental import pallas as pl
from jax.experimental.pallas import tpu as pltpu
from jax.experimental.pallas import tpu_sc as plsc

_N = 320000
_D = 128
_B = 10000
_BN = 3200

_NW = 32
_BPW = 320
_BPAD = _NW * _BPW
_WS = 16384
_VPW = _WS // 16
_SSTRIDE = 2560
_NSAMP = _N // _SSTRIDE


def _matvec_body(x_ref, w_ref, b_ref, o_ref):
    i = pl.program_id(0)
    o_ref[pl.ds(pl.multiple_of(i * _BN, 128), _BN)] = (
        jnp.dot(x_ref[...].astype(jnp.bfloat16), w_ref[...].astype(jnp.bfloat16).T,
                preferred_element_type=jnp.float32)[:, 0]
        + b_ref[0, 0])


def _matvec(z_ins, W, b):
    n = z_ins.shape[0]
    grid = n // _BN
    return pl.pallas_call(
        _matvec_body,
        grid=(grid,),
        in_specs=[
            pl.BlockSpec((_BN, _D), lambda i: (i, 0)),
            pl.BlockSpec((1, _D), lambda i: (0, 0)),
            pl.BlockSpec((1, 1), lambda i: (0, 0)),
        ],
        out_specs=pl.BlockSpec((n,), lambda i: (0,)),
        out_shape=jax.ShapeDtypeStruct((n,), jnp.float32),
    )(z_ins, W, b.reshape(1, 1))


@functools.partial(
    pl.kernel,
    out_type=[
        jax.ShapeDtypeStruct((_BPAD,), jnp.float32),
        jax.ShapeDtypeStruct((_BPAD, _D), jnp.float32),
        jax.ShapeDtypeStruct((_BPAD, _D), jnp.float32),
        jax.ShapeDtypeStruct((_BPAD, _D), jnp.float32),
        jax.ShapeDtypeStruct((_BPAD, _D), jnp.float32),
    ],
    mesh=plsc.VectorSubcoreMesh(core_axis_name="c", subcore_axis_name="s"),
    compiler_params=pltpu.CompilerParams(needs_layout_passes=False),
    scratch_types=[
        pltpu.VMEM((128,), jnp.int32),
        pltpu.VMEM((128,), jnp.int32),
        pltpu.VMEM((_SSTRIDE,), jnp.int32),
        pltpu.VMEM((_WS,), jnp.int32),
        pltpu.VMEM((_WS,), jnp.float32),
        pltpu.VMEM((_BPW,), jnp.float32),
        pltpu.VMEM((_BPW,), jnp.int32),
        pltpu.VMEM((128, _D), jnp.float32),
        pltpu.SemaphoreType.DMA,
    ],
)
def _sc_segment(score_hbm, bag_hbm, t0_hbm, t1_hbm, t2_hbm, t3_hbm,
                m_hbm, o0_hbm, o1_hbm, o2_hbm, o3_hbm,
                sidx_v, samp_v, srch_v, bag_w, score_w, m_loc, loc_loc,
                rows_v, sem):
    wid = lax.axis_index("s") * 2 + lax.axis_index("c")
    blo = wid * _BPW
    t_lo = jnp.minimum(blo, _B)
    t_hi = jnp.minimum(blo + _BPW, _B)

    iota = lax.iota(jnp.int32, 16)
    zeros_i = jnp.zeros((16,), jnp.int32)
    ninf = jnp.full((16,), -jnp.inf, jnp.float32)
    imax = jnp.int32(2**31 - 1)
    bigbag = jnp.int32(2**30)

    for k in range(8):
        lane = iota + 16 * k
        sidx_v[pl.ds(16 * k, 16)] = jnp.where(lane < _NSAMP, lane * _SSTRIDE, 0)
    pltpu.async_copy(bag_hbm.at[sidx_v], samp_v, sem).wait()

    def lower_bound(t):
        def cbody(k, c):
            sv = samp_v[pl.ds(16 * k, 16)]
            lane = iota + 16 * k
            return c + plsc.all_reduce_population_count(
                (lane < _NSAMP) & (sv < t))[0]
        c = lax.fori_loop(0, 8, cbody, jnp.int32(0))
        base = _SSTRIDE * jnp.maximum(c - 1, 0)
        pltpu.sync_copy(bag_hbm.at[pl.ds(pl.multiple_of(base, 8), _SSTRIDE)], srch_v)
        def fbody(k, c2):
            sv = srch_v[pl.ds(16 * k, 16)]
            return c2 + plsc.all_reduce_population_count(sv < t)[0]
        c2 = lax.fori_loop(0, _SSTRIDE // 16, fbody, jnp.int32(0))
        return base + c2

    r0 = lower_bound(t_lo)
    r1 = lower_bound(t_hi)

    for k in range(_BPW // 16):
        loc_loc[pl.ds(16 * k, 16)] = zeros_i

    lane0 = iota == 0

    def emit(j, mv, loc):
        jc = jnp.broadcast_to(jnp.minimum(j, _BPW - 1), (16,))
        plsc.store_scatter(m_loc, [jc], jnp.broadcast_to(mv, (16,)), mask=lane0)
        plsc.store_scatter(loc_loc, [jc], jnp.broadcast_to(loc, (16,)),
                           mask=lane0)

    _gdn = lax.GatherDimensionNumbers(
        offset_dims=(), collapsed_slice_dims=(0,), start_index_map=(0,))

    def splat_lane(vec, lane):
        idx = jnp.broadcast_to(lane, (16,)).astype(jnp.int32)[:, None]
        return lax.gather(vec, idx, _gdn, (1,),
                          mode=lax.GatherScatterMode.PROMISE_IN_BOUNDS)

    @pl.when(r1 > r0)
    def _():
        v0 = r0 - (r0 % 16)
        nv = (r1 - v0 + 15) // 16

        def vbody(k, carry):
            b, s_b, j, m, idx, done = carry
            win_base = pl.multiple_of(jnp.minimum(v0 + 16 * (k - k % _VPW), _N - _WS), 16)

            @pl.when(k % _VPW == 0)
            def _():
                pltpu.sync_copy(bag_hbm.at[pl.ds(win_base, _WS)], bag_w)
                pltpu.sync_copy(score_hbm.at[pl.ds(win_base, _WS)], score_w)

            p = v0 + 16 * k
            rel = p - win_base
            g = bag_w[pl.ds(rel, 16)]
            s = score_w[pl.ds(rel, 16)]
            p_lane = p + iota
            g_eff = jnp.where(p_lane >= r1, bigbag, g)
            off0 = jnp.where(k == 0, r0 - v0, jnp.int32(0))

            b, s_b = lax.cond(
                k == 0,
                lambda b, s_b: (splat_lane(g, off0), r0),
                lambda b, s_b: (b, s_b),
                b, s_b)

            def wcond(st):
                off, _b, _sb, _j, _m, _i, dn = st
                return (off < 16) & (dn == 0)

            def wbody(st):
                off, b, s_b, j, m, idx, done = st
                active = (iota >= off) & (g_eff == b)
                cnt = plsc.all_reduce_population_count(active)[0]
                upd = active & (s > m)
                m2 = jnp.where(upd, s, m)
                idx2 = jnp.where(upd, p_lane, idx)
                off2 = off + cnt

                def fin(off2, b, s_b, j, m2, idx2):
                    mv = jnp.max(m2)
                    cand = jnp.where(m2 == mv, idx2, imax)
                    gm = jnp.min(cand)
                    emit(j, mv, gm - s_b)
                    p_next = p + off2
                    done2 = jnp.where(p_next >= r1, jnp.int32(1), jnp.int32(0))
                    b2 = splat_lane(g, jnp.minimum(off2, 15))
                    return (off2, b2, p_next, j + 1, ninf, zeros_i, done2)

                def cont(off2, b, s_b, j, m2, idx2):
                    return (off2, b, s_b, j, m2, idx2, jnp.int32(0))

                return lax.cond(off2 < 16, fin, cont, off2, b, s_b, j, m2, idx2)

            st = lax.while_loop(wcond, wbody, (off0, b, s_b, j, m, idx, done))
            return st[1:]

        init = (zeros_i, jnp.int32(0), jnp.int32(0), ninf, zeros_i,
                jnp.int32(0))
        b, s_b, j, m, idx, done = lax.fori_loop(0, nv, vbody, init)

        @pl.when(done == 0)
        def _():
            mv = jnp.max(m)
            cand = jnp.where(m == mv, idx, imax)
            gm = jnp.min(cand)
            emit(j, mv, gm - s_b)

    pltpu.sync_copy(m_loc, m_hbm.at[pl.ds(pl.multiple_of(blo, 8), _BPW)])
    for tbl, out in ((t0_hbm, o0_hbm), (t1_hbm, o1_hbm),
                     (t2_hbm, o2_hbm), (t3_hbm, o3_hbm)):
        for c0, csz in ((0, 128), (128, 128), (256, 64)):
            pltpu.async_copy(tbl.at[loc_loc.at[pl.ds(c0, csz)]],
                             rows_v.at[pl.ds(0, csz)], sem).wait()
            pltpu.sync_copy(rows_v.at[pl.ds(0, csz)],
                            out.at[pl.ds(pl.multiple_of(blo + c0, 8), csz)])


def kernel(z_ins, bag_idx, bag_instances, instance_mu, instance_std, W, b):
    score = _matvec(z_ins, W, b)
    m_pad, o_z, o_inst, o_mu, o_std = _sc_segment(
        score, bag_idx, z_ins, bag_instances, instance_mu, instance_std)
    return (m_pad[:_B, None], o_inst[:_B], o_z[:_B], score[:, None],
            o_mu[:_B], o_std[:_B])


# speedup vs baseline: 83.1245x; 1.0450x over previous
import functools

import jax
import jax.numpy as jnp
from jax import lax
from jax.experimental import pallas as pl
from jax.experimental.pallas import tpu as pltpu
from jax.experimental.pallas import tpu_sc as plsc

_N = 320000
_D = 128
_B = 10000
_BN = 12800

_NW = 32
_BPW = 320
_BPAD = _NW * _BPW
_WS = 16384
_VPW = _WS // 16
_SSTRIDE = 2560
_NSAMP = _N // _SSTRIDE


def _matvec_body(x_ref, w_ref, b_ref, o_ref):
    i = pl.program_id(0)
    o_ref[pl.ds(pl.multiple_of(i * _BN, 128), _BN)] = (
        jnp.dot(x_ref[...].astype(jnp.bfloat16), w_ref[...].astype(jnp.bfloat16).T,
                preferred_element_type=jnp.float32)[:, 0]
        + b_ref[0, 0])


def _matvec(z_ins, W, b):
    n = z_ins.shape[0]
    grid = n // _BN
    return pl.pallas_call(
        _matvec_body,
        grid=(grid,),
        in_specs=[
            pl.BlockSpec((_BN, _D), lambda i: (i, 0)),
            pl.BlockSpec((1, _D), lambda i: (0, 0)),
            pl.BlockSpec((1, 1), lambda i: (0, 0)),
        ],
        out_specs=pl.BlockSpec((n,), lambda i: (0,)),
        out_shape=jax.ShapeDtypeStruct((n,), jnp.float32),
    )(z_ins, W, b.reshape(1, 1))


@functools.partial(
    pl.kernel,
    out_type=[
        jax.ShapeDtypeStruct((_BPAD,), jnp.float32),
        jax.ShapeDtypeStruct((_BPAD, _D), jnp.float32),
        jax.ShapeDtypeStruct((_BPAD, _D), jnp.float32),
        jax.ShapeDtypeStruct((_BPAD, _D), jnp.float32),
        jax.ShapeDtypeStruct((_BPAD, _D), jnp.float32),
    ],
    mesh=plsc.VectorSubcoreMesh(core_axis_name="c", subcore_axis_name="s"),
    compiler_params=pltpu.CompilerParams(needs_layout_passes=False),
    scratch_types=[
        pltpu.VMEM((128,), jnp.int32),
        pltpu.VMEM((128,), jnp.int32),
        pltpu.VMEM((_SSTRIDE,), jnp.int32),
        pltpu.VMEM((_WS,), jnp.int32),
        pltpu.VMEM((_WS,), jnp.float32),
        pltpu.VMEM((_BPW,), jnp.float32),
        pltpu.VMEM((_BPW,), jnp.int32),
        pltpu.VMEM((128, _D), jnp.float32),
        pltpu.SemaphoreType.DMA,
    ],
)
def _sc_segment(score_hbm, bag_hbm, t0_hbm, t1_hbm, t2_hbm, t3_hbm,
                m_hbm, o0_hbm, o1_hbm, o2_hbm, o3_hbm,
                sidx_v, samp_v, srch_v, bag_w, score_w, m_loc, loc_loc,
                rows_v, sem):
    wid = lax.axis_index("s") * 2 + lax.axis_index("c")
    blo = wid * _BPW
    t_lo = jnp.minimum(blo, _B)
    t_hi = jnp.minimum(blo + _BPW, _B)

    iota = lax.iota(jnp.int32, 16)
    zeros_i = jnp.zeros((16,), jnp.int32)
    ninf = jnp.full((16,), -jnp.inf, jnp.float32)
    imax = jnp.int32(2**31 - 1)
    bigbag = jnp.int32(2**30)

    for k in range(8):
        lane = iota + 16 * k
        sidx_v[pl.ds(16 * k, 16)] = jnp.where(lane < _NSAMP, lane * _SSTRIDE, 0)
    pltpu.async_copy(bag_hbm.at[sidx_v], samp_v, sem).wait()

    def lower_bound(t):
        def cbody(k, c):
            sv = samp_v[pl.ds(16 * k, 16)]
            lane = iota + 16 * k
            return c + plsc.all_reduce_population_count(
                (lane < _NSAMP) & (sv < t))[0]
        c = lax.fori_loop(0, 8, cbody, jnp.int32(0))
        base = _SSTRIDE * jnp.maximum(c - 1, 0)
        pltpu.sync_copy(bag_hbm.at[pl.ds(pl.multiple_of(base, 8), _SSTRIDE)], srch_v)
        def fbody(k, c2):
            sv = srch_v[pl.ds(16 * k, 16)]
            return c2 + plsc.all_reduce_population_count(sv < t)[0]
        c2 = lax.fori_loop(0, _SSTRIDE // 16, fbody, jnp.int32(0))
        return base + c2

    r0 = lower_bound(t_lo)
    r1 = lower_bound(t_hi)

    for k in range(_BPW // 16):
        loc_loc[pl.ds(16 * k, 16)] = zeros_i

    lane0 = iota == 0

    def emit(j, mv, loc):
        jc = jnp.broadcast_to(jnp.minimum(j, _BPW - 1), (16,))
        plsc.store_scatter(m_loc, [jc], jnp.broadcast_to(mv, (16,)), mask=lane0)
        plsc.store_scatter(loc_loc, [jc], jnp.broadcast_to(loc, (16,)),
                           mask=lane0)

    _gdn = lax.GatherDimensionNumbers(
        offset_dims=(), collapsed_slice_dims=(0,), start_index_map=(0,))

    def splat_lane(vec, lane):
        idx = jnp.broadcast_to(lane, (16,)).astype(jnp.int32)[:, None]
        return lax.gather(vec, idx, _gdn, (1,),
                          mode=lax.GatherScatterMode.PROMISE_IN_BOUNDS)

    @pl.when(r1 > r0)
    def _():
        v0 = r0 - (r0 % 16)
        nv = (r1 - v0 + 15) // 16

        def vbody(k, carry):
            b, s_b, j, m, idx, done = carry
            win_base = pl.multiple_of(jnp.minimum(v0 + 16 * (k - k % _VPW), _N - _WS), 16)

            @pl.when(k % _VPW == 0)
            def _():
                pltpu.sync_copy(bag_hbm.at[pl.ds(win_base, _WS)], bag_w)
                pltpu.sync_copy(score_hbm.at[pl.ds(win_base, _WS)], score_w)

            p = v0 + 16 * k
            rel = p - win_base
            g = bag_w[pl.ds(rel, 16)]
            s = score_w[pl.ds(rel, 16)]
            p_lane = p + iota
            g_eff = jnp.where(p_lane >= r1, bigbag, g)
            off0 = jnp.where(k == 0, r0 - v0, jnp.int32(0))

            b, s_b = lax.cond(
                k == 0,
                lambda b, s_b: (splat_lane(g, off0), r0),
                lambda b, s_b: (b, s_b),
                b, s_b)

            def wcond(st):
                off, _b, _sb, _j, _m, _i, dn = st
                return (off < 16) & (dn == 0)

            def wbody(st):
                off, b, s_b, j, m, idx, done = st
                active = (iota >= off) & (g_eff == b)
                cnt = plsc.all_reduce_population_count(active)[0]
                upd = active & (s > m)
                m2 = jnp.where(upd, s, m)
                idx2 = jnp.where(upd, p_lane, idx)
                off2 = off + cnt

                def fin(off2, b, s_b, j, m2, idx2):
                    mv = jnp.max(m2)
                    cand = jnp.where(m2 == mv, idx2, imax)
                    gm = jnp.min(cand)
                    emit(j, mv, gm - s_b)
                    p_next = p + off2
                    done2 = jnp.where(p_next >= r1, jnp.int32(1), jnp.int32(0))
                    b2 = splat_lane(g, jnp.minimum(off2, 15))
                    return (off2, b2, p_next, j + 1, ninf, zeros_i, done2)

                def cont(off2, b, s_b, j, m2, idx2):
                    return (off2, b, s_b, j, m2, idx2, jnp.int32(0))

                return lax.cond(off2 < 16, fin, cont, off2, b, s_b, j, m2, idx2)

            st = lax.while_loop(wcond, wbody, (off0, b, s_b, j, m, idx, done))
            return st[1:]

        init = (zeros_i, jnp.int32(0), jnp.int32(0), ninf, zeros_i,
                jnp.int32(0))
        b, s_b, j, m, idx, done = lax.fori_loop(0, nv, vbody, init)

        @pl.when(done == 0)
        def _():
            mv = jnp.max(m)
            cand = jnp.where(m == mv, idx, imax)
            gm = jnp.min(cand)
            emit(j, mv, gm - s_b)

    pltpu.sync_copy(m_loc, m_hbm.at[pl.ds(pl.multiple_of(blo, 8), _BPW)])
    for tbl, out in ((t0_hbm, o0_hbm), (t1_hbm, o1_hbm),
                     (t2_hbm, o2_hbm), (t3_hbm, o3_hbm)):
        for c0, csz in ((0, 128), (128, 128), (256, 64)):
            pltpu.async_copy(tbl.at[loc_loc.at[pl.ds(c0, csz)]],
                             rows_v.at[pl.ds(0, csz)], sem).wait()
            pltpu.sync_copy(rows_v.at[pl.ds(0, csz)],
                            out.at[pl.ds(pl.multiple_of(blo + c0, 8), csz)])


def kernel(z_ins, bag_idx, bag_instances, instance_mu, instance_std, W, b):
    score = _matvec(z_ins, W, b)
    m_pad, o_z, o_inst, o_mu, o_std = _sc_segment(
        score, bag_idx, z_ins, bag_instances, instance_mu, instance_std)
    return (m_pad[:_B, None], o_inst[:_B], o_z[:_B], score[:, None],
            o_mu[:_B], o_std[:_B])


# speedup vs baseline: 120.7167x; 1.4522x over previous
import functools

import jax
import jax.numpy as jnp
from jax import lax
from jax.experimental import pallas as pl
from jax.experimental.pallas import tpu as pltpu
from jax.experimental.pallas import tpu_sc as plsc

_N = 320000
_D = 128
_B = 10000
_BN = 12800

_NW = 32
_BPW = 320
_BPAD = _NW * _BPW
_WS = 16384
_VPW = _WS // 16
_SSTRIDE = 2560
_NSAMP = _N // _SSTRIDE


def _matvec_body(x_ref, w_ref, b_ref, o_ref):
    i = pl.program_id(0)
    prod = lax.dot_general(
        w_ref[...].astype(jnp.bfloat16), x_ref[...].astype(jnp.bfloat16),
        (((1,), (1,)), ((), ())), preferred_element_type=jnp.float32)
    o_ref[pl.ds(pl.multiple_of(i * _BN, 128), _BN)] = prod[0] + b_ref[0, 0]


def _matvec(z_ins, W, b):
    n = z_ins.shape[0]
    grid = n // _BN
    return pl.pallas_call(
        _matvec_body,
        grid=(grid,),
        in_specs=[
            pl.BlockSpec((_BN, _D), lambda i: (i, 0)),
            pl.BlockSpec((1, _D), lambda i: (0, 0)),
            pl.BlockSpec((1, 1), lambda i: (0, 0)),
        ],
        out_specs=pl.BlockSpec((n,), lambda i: (0,)),
        out_shape=jax.ShapeDtypeStruct((n,), jnp.float32),
    )(z_ins, W, b.reshape(1, 1))


@functools.partial(
    pl.kernel,
    out_type=[
        jax.ShapeDtypeStruct((_BPAD,), jnp.float32),
        jax.ShapeDtypeStruct((_BPAD, _D), jnp.float32),
        jax.ShapeDtypeStruct((_BPAD, _D), jnp.float32),
        jax.ShapeDtypeStruct((_BPAD, _D), jnp.float32),
        jax.ShapeDtypeStruct((_BPAD, _D), jnp.float32),
    ],
    mesh=plsc.VectorSubcoreMesh(core_axis_name="c", subcore_axis_name="s"),
    compiler_params=pltpu.CompilerParams(needs_layout_passes=False),
    scratch_types=[
        pltpu.VMEM((128,), jnp.int32),
        pltpu.VMEM((128,), jnp.int32),
        pltpu.VMEM((_SSTRIDE,), jnp.int32),
        pltpu.VMEM((_WS,), jnp.int32),
        pltpu.VMEM((_WS,), jnp.float32),
        pltpu.VMEM((_BPW,), jnp.float32),
        pltpu.VMEM((_BPW,), jnp.int32),
        pltpu.VMEM((128, _D), jnp.float32),
        pltpu.SemaphoreType.DMA,
    ],
)
def _sc_segment(score_hbm, bag_hbm, t0_hbm, t1_hbm, t2_hbm, t3_hbm,
                m_hbm, o0_hbm, o1_hbm, o2_hbm, o3_hbm,
                sidx_v, samp_v, srch_v, bag_w, score_w, m_loc, loc_loc,
                rows_v, sem):
    wid = lax.axis_index("s") * 2 + lax.axis_index("c")
    blo = wid * _BPW
    t_lo = jnp.minimum(blo, _B)
    t_hi = jnp.minimum(blo + _BPW, _B)

    iota = lax.iota(jnp.int32, 16)
    zeros_i = jnp.zeros((16,), jnp.int32)
    ninf = jnp.full((16,), -jnp.inf, jnp.float32)
    imax = jnp.int32(2**31 - 1)
    bigbag = jnp.int32(2**30)

    for k in range(8):
        lane = iota + 16 * k
        sidx_v[pl.ds(16 * k, 16)] = jnp.where(lane < _NSAMP, lane * _SSTRIDE, 0)
    pltpu.async_copy(bag_hbm.at[sidx_v], samp_v, sem).wait()

    def lower_bound(t):
        def cbody(k, c):
            sv = samp_v[pl.ds(16 * k, 16)]
            lane = iota + 16 * k
            return c + plsc.all_reduce_population_count(
                (lane < _NSAMP) & (sv < t))[0]
        c = lax.fori_loop(0, 8, cbody, jnp.int32(0))
        base = _SSTRIDE * jnp.maximum(c - 1, 0)
        pltpu.sync_copy(bag_hbm.at[pl.ds(pl.multiple_of(base, 8), _SSTRIDE)], srch_v)
        def fbody(k, c2):
            sv = srch_v[pl.ds(16 * k, 16)]
            return c2 + plsc.all_reduce_population_count(sv < t)[0]
        c2 = lax.fori_loop(0, _SSTRIDE // 16, fbody, jnp.int32(0))
        return base + c2

    r0 = lower_bound(t_lo)
    r1 = lower_bound(t_hi)

    for k in range(_BPW // 16):
        loc_loc[pl.ds(16 * k, 16)] = zeros_i

    lane0 = iota == 0

    def emit(j, mv, loc):
        jc = jnp.broadcast_to(jnp.minimum(j, _BPW - 1), (16,))
        plsc.store_scatter(m_loc, [jc], jnp.broadcast_to(mv, (16,)), mask=lane0)
        plsc.store_scatter(loc_loc, [jc], jnp.broadcast_to(loc, (16,)),
                           mask=lane0)

    _gdn = lax.GatherDimensionNumbers(
        offset_dims=(), collapsed_slice_dims=(0,), start_index_map=(0,))

    def splat_lane(vec, lane):
        idx = jnp.broadcast_to(lane, (16,)).astype(jnp.int32)[:, None]
        return lax.gather(vec, idx, _gdn, (1,),
                          mode=lax.GatherScatterMode.PROMISE_IN_BOUNDS)

    @pl.when(r1 > r0)
    def _():
        v0 = r0 - (r0 % 16)
        nv = (r1 - v0 + 15) // 16

        def vbody(k, carry):
            b, s_b, j, m, idx, done = carry
            win_base = pl.multiple_of(jnp.minimum(v0 + 16 * (k - k % _VPW), _N - _WS), 16)

            @pl.when(k % _VPW == 0)
            def _():
                pltpu.sync_copy(bag_hbm.at[pl.ds(win_base, _WS)], bag_w)
                pltpu.sync_copy(score_hbm.at[pl.ds(win_base, _WS)], score_w)

            p = v0 + 16 * k
            rel = p - win_base
            g = bag_w[pl.ds(rel, 16)]
            s = score_w[pl.ds(rel, 16)]
            p_lane = p + iota
            g_eff = jnp.where(p_lane >= r1, bigbag, g)
            off0 = jnp.where(k == 0, r0 - v0, jnp.int32(0))

            b, s_b = lax.cond(
                k == 0,
                lambda b, s_b: (splat_lane(g, off0), r0),
                lambda b, s_b: (b, s_b),
                b, s_b)

            def wcond(st):
                off, _b, _sb, _j, _m, _i, dn = st
                return (off < 16) & (dn == 0)

            def wbody(st):
                off, b, s_b, j, m, idx, done = st
                active = (iota >= off) & (g_eff == b)
                cnt = plsc.all_reduce_population_count(active)[0]
                upd = active & (s > m)
                m2 = jnp.where(upd, s, m)
                idx2 = jnp.where(upd, p_lane, idx)
                off2 = off + cnt

                def fin(off2, b, s_b, j, m2, idx2):
                    mv = jnp.max(m2)
                    cand = jnp.where(m2 == mv, idx2, imax)
                    gm = jnp.min(cand)
                    emit(j, mv, gm - s_b)
                    p_next = p + off2
                    done2 = jnp.where(p_next >= r1, jnp.int32(1), jnp.int32(0))
                    b2 = splat_lane(g, jnp.minimum(off2, 15))
                    return (off2, b2, p_next, j + 1, ninf, zeros_i, done2)

                def cont(off2, b, s_b, j, m2, idx2):
                    return (off2, b, s_b, j, m2, idx2, jnp.int32(0))

                return lax.cond(off2 < 16, fin, cont, off2, b, s_b, j, m2, idx2)

            st = lax.while_loop(wcond, wbody, (off0, b, s_b, j, m, idx, done))
            return st[1:]

        init = (zeros_i, jnp.int32(0), jnp.int32(0), ninf, zeros_i,
                jnp.int32(0))
        b, s_b, j, m, idx, done = lax.fori_loop(0, nv, vbody, init)

        @pl.when(done == 0)
        def _():
            mv = jnp.max(m)
            cand = jnp.where(m == mv, idx, imax)
            gm = jnp.min(cand)
            emit(j, mv, gm - s_b)

    pltpu.sync_copy(m_loc, m_hbm.at[pl.ds(pl.multiple_of(blo, 8), _BPW)])
    for tbl, out in ((t0_hbm, o0_hbm), (t1_hbm, o1_hbm),
                     (t2_hbm, o2_hbm), (t3_hbm, o3_hbm)):
        for c0, csz in ((0, 128), (128, 128), (256, 64)):
            pltpu.async_copy(tbl.at[loc_loc.at[pl.ds(c0, csz)]],
                             rows_v.at[pl.ds(0, csz)], sem).wait()
            pltpu.sync_copy(rows_v.at[pl.ds(0, csz)],
                            out.at[pl.ds(pl.multiple_of(blo + c0, 8), csz)])


def kernel(z_ins, bag_idx, bag_instances, instance_mu, instance_std, W, b):
    score = _matvec(z_ins, W, b)
    m_pad, o_z, o_inst, o_mu, o_std = _sc_segment(
        score, bag_idx, z_ins, bag_instances, instance_mu, instance_std)
    return (m_pad[:_B, None], o_inst[:_B], o_z[:_B], score[:, None],
            o_mu[:_B], o_std[:_B])


# speedup vs baseline: 127.6116x; 1.0571x over previous
import functools

import jax
import jax.numpy as jnp
from jax import lax
from jax.experimental import pallas as pl
from jax.experimental.pallas import tpu as pltpu
from jax.experimental.pallas import tpu_sc as plsc

_N = 320000
_D = 128
_B = 10000
_BN = 12800

_NW = 32
_BPW = 320
_BPAD = _NW * _BPW
_WS = 16384
_VPW = _WS // 16
_SSTRIDE = 2560
_NSAMP = _N // _SSTRIDE


def _matvec_body(x_ref, w_ref, b_ref, o_ref):
    i = pl.program_id(0)
    prod = lax.dot_general(
        w_ref[...].astype(jnp.bfloat16), x_ref[...].astype(jnp.bfloat16),
        (((1,), (1,)), ((), ())), preferred_element_type=jnp.float32)
    o_ref[pl.ds(pl.multiple_of(i * _BN, 128), _BN)] = prod[0] + b_ref[0, 0]


def _matvec(z_ins, W, b):
    n = z_ins.shape[0]
    grid = n // _BN
    return pl.pallas_call(
        _matvec_body,
        grid=(grid,),
        in_specs=[
            pl.BlockSpec((_BN, _D), lambda i: (i, 0)),
            pl.BlockSpec((1, _D), lambda i: (0, 0)),
            pl.BlockSpec((1, 1), lambda i: (0, 0)),
        ],
        out_specs=pl.BlockSpec((n,), lambda i: (0,)),
        out_shape=jax.ShapeDtypeStruct((n,), jnp.float32),
    )(z_ins, W, b.reshape(1, 1))


@functools.partial(
    pl.kernel,
    out_type=[
        jax.ShapeDtypeStruct((_BPAD,), jnp.float32),
        jax.ShapeDtypeStruct((_BPAD, _D), jnp.float32),
        jax.ShapeDtypeStruct((_BPAD, _D), jnp.float32),
        jax.ShapeDtypeStruct((_BPAD, _D), jnp.float32),
        jax.ShapeDtypeStruct((_BPAD, _D), jnp.float32),
    ],
    mesh=plsc.VectorSubcoreMesh(core_axis_name="c", subcore_axis_name="s"),
    compiler_params=pltpu.CompilerParams(needs_layout_passes=False),
    scratch_types=[
        pltpu.VMEM((128,), jnp.int32),
        pltpu.VMEM((128,), jnp.int32),
        pltpu.VMEM((_SSTRIDE,), jnp.int32),
        pltpu.VMEM((_WS,), jnp.int32),
        pltpu.VMEM((_WS,), jnp.float32),
        pltpu.VMEM((_BPW,), jnp.float32),
        pltpu.VMEM((_BPW,), jnp.int32),
        pltpu.VMEM((_BPW, _D), jnp.float32),
        pltpu.SemaphoreType.DMA,
    ],
)
def _sc_segment(score_hbm, bag_hbm, t0_hbm, t1_hbm, t2_hbm, t3_hbm,
                m_hbm, o0_hbm, o1_hbm, o2_hbm, o3_hbm,
                sidx_v, samp_v, srch_v, bag_w, score_w, m_loc, loc_loc,
                rows_v, sem):
    wid = lax.axis_index("s") * 2 + lax.axis_index("c")
    blo = wid * _BPW
    t_lo = jnp.minimum(blo, _B)
    t_hi = jnp.minimum(blo + _BPW, _B)

    iota = lax.iota(jnp.int32, 16)
    zeros_i = jnp.zeros((16,), jnp.int32)
    ninf = jnp.full((16,), -jnp.inf, jnp.float32)
    imax = jnp.int32(2**31 - 1)
    bigbag = jnp.int32(2**30)

    for k in range(8):
        lane = iota + 16 * k
        sidx_v[pl.ds(16 * k, 16)] = jnp.where(lane < _NSAMP, lane * _SSTRIDE, 0)
    pltpu.async_copy(bag_hbm.at[sidx_v], samp_v, sem).wait()

    def lower_bound(t):
        def cbody(k, c):
            sv = samp_v[pl.ds(16 * k, 16)]
            lane = iota + 16 * k
            return c + plsc.all_reduce_population_count(
                (lane < _NSAMP) & (sv < t))[0]
        c = lax.fori_loop(0, 8, cbody, jnp.int32(0))
        base = _SSTRIDE * jnp.maximum(c - 1, 0)
        pltpu.sync_copy(bag_hbm.at[pl.ds(pl.multiple_of(base, 8), _SSTRIDE)], srch_v)
        def fbody(k, c2):
            sv = srch_v[pl.ds(16 * k, 16)]
            return c2 + plsc.all_reduce_population_count(sv < t)[0]
        c2 = lax.fori_loop(0, _SSTRIDE // 16, fbody, jnp.int32(0))
        return base + c2

    r0 = lower_bound(t_lo)
    r1 = lower_bound(t_hi)

    for k in range(_BPW // 16):
        loc_loc[pl.ds(16 * k, 16)] = zeros_i

    lane0 = iota == 0

    def emit(j, mv, loc):
        jc = jnp.broadcast_to(jnp.minimum(j, _BPW - 1), (16,))
        plsc.store_scatter(m_loc, [jc], jnp.broadcast_to(mv, (16,)), mask=lane0)
        plsc.store_scatter(loc_loc, [jc], jnp.broadcast_to(loc, (16,)),
                           mask=lane0)

    _gdn = lax.GatherDimensionNumbers(
        offset_dims=(), collapsed_slice_dims=(0,), start_index_map=(0,))

    def splat_lane(vec, lane):
        idx = jnp.broadcast_to(lane, (16,)).astype(jnp.int32)[:, None]
        return lax.gather(vec, idx, _gdn, (1,),
                          mode=lax.GatherScatterMode.PROMISE_IN_BOUNDS)

    @pl.when(r1 > r0)
    def _():
        v0 = r0 - (r0 % 16)
        nv = (r1 - v0 + 15) // 16

        def vbody(k, carry):
            b, s_b, j, m, idx, done = carry
            win_base = pl.multiple_of(jnp.minimum(v0 + 16 * (k - k % _VPW), _N - _WS), 16)

            @pl.when(k % _VPW == 0)
            def _():
                pltpu.sync_copy(bag_hbm.at[pl.ds(win_base, _WS)], bag_w)
                pltpu.sync_copy(score_hbm.at[pl.ds(win_base, _WS)], score_w)

            p = v0 + 16 * k
            rel = p - win_base
            g = bag_w[pl.ds(rel, 16)]
            s = score_w[pl.ds(rel, 16)]
            p_lane = p + iota
            g_eff = jnp.where(p_lane >= r1, bigbag, g)
            off0 = jnp.where(k == 0, r0 - v0, jnp.int32(0))

            b, s_b = lax.cond(
                k == 0,
                lambda b, s_b: (splat_lane(g, off0), r0),
                lambda b, s_b: (b, s_b),
                b, s_b)

            def wcond(st):
                off, _b, _sb, _j, _m, _i, dn = st
                return (off < 16) & (dn == 0)

            def wbody(st):
                off, b, s_b, j, m, idx, done = st
                active = (iota >= off) & (g_eff == b)
                cnt = plsc.all_reduce_population_count(active)[0]
                upd = active & (s > m)
                m2 = jnp.where(upd, s, m)
                idx2 = jnp.where(upd, p_lane, idx)
                off2 = off + cnt

                def fin(off2, b, s_b, j, m2, idx2):
                    mv = jnp.max(m2)
                    cand = jnp.where(m2 == mv, idx2, imax)
                    gm = jnp.min(cand)
                    emit(j, mv, gm - s_b)
                    p_next = p + off2
                    done2 = jnp.where(p_next >= r1, jnp.int32(1), jnp.int32(0))
                    b2 = splat_lane(g, jnp.minimum(off2, 15))
                    return (off2, b2, p_next, j + 1, ninf, zeros_i, done2)

                def cont(off2, b, s_b, j, m2, idx2):
                    return (off2, b, s_b, j, m2, idx2, jnp.int32(0))

                return lax.cond(off2 < 16, fin, cont, off2, b, s_b, j, m2, idx2)

            st = lax.while_loop(wcond, wbody, (off0, b, s_b, j, m, idx, done))
            return st[1:]

        init = (zeros_i, jnp.int32(0), jnp.int32(0), ninf, zeros_i,
                jnp.int32(0))
        b, s_b, j, m, idx, done = lax.fori_loop(0, nv, vbody, init)

        @pl.when(done == 0)
        def _():
            mv = jnp.max(m)
            cand = jnp.where(m == mv, idx, imax)
            gm = jnp.min(cand)
            emit(j, mv, gm - s_b)

    pltpu.sync_copy(m_loc, m_hbm.at[pl.ds(pl.multiple_of(blo, 8), _BPW)])
    for tbl, out in ((t0_hbm, o0_hbm), (t1_hbm, o1_hbm),
                     (t2_hbm, o2_hbm), (t3_hbm, o3_hbm)):
        copies = [
            pltpu.async_copy(tbl.at[loc_loc.at[pl.ds(c0, csz)]],
                             rows_v.at[pl.ds(c0, csz)], sem)
            for c0, csz in ((0, 128), (128, 128), (256, 64))]
        for c in copies:
            c.wait()
        pltpu.sync_copy(rows_v, out.at[pl.ds(pl.multiple_of(blo, 8), _BPW)])


def kernel(z_ins, bag_idx, bag_instances, instance_mu, instance_std, W, b):
    score = _matvec(z_ins, W, b)
    m_pad, o_z, o_inst, o_mu, o_std = _sc_segment(
        score, bag_idx, z_ins, bag_instances, instance_mu, instance_std)
    return (m_pad[:_B, None], o_inst[:_B], o_z[:_B], score[:, None],
            o_mu[:_B], o_std[:_B])


# speedup vs baseline: 127.9499x; 1.0027x over previous
import functools

import jax
import jax.numpy as jnp
from jax import lax
from jax.experimental import pallas as pl
from jax.experimental.pallas import tpu as pltpu
from jax.experimental.pallas import tpu_sc as plsc

_N = 320000
_D = 128
_B = 10000
_BN = 12800

_NW = 32
_BPW = 320
_BPAD = _NW * _BPW
_WS = 16384
_VPW = _WS // 16
_SSTRIDE = 2560
_NSAMP = _N // _SSTRIDE


def _matvec_body(x_ref, w_ref, b_ref, o_ref):
    i = pl.program_id(0)
    prod = lax.dot_general(
        w_ref[...].astype(jnp.bfloat16), x_ref[...].astype(jnp.bfloat16),
        (((1,), (1,)), ((), ())), preferred_element_type=jnp.float32)
    o_ref[pl.ds(pl.multiple_of(i * _BN, 128), _BN)] = prod[0] + b_ref[0, 0]


def _matvec(z_ins, W, b):
    n = z_ins.shape[0]
    grid = n // _BN
    return pl.pallas_call(
        _matvec_body,
        grid=(grid,),
        in_specs=[
            pl.BlockSpec((_BN, _D), lambda i: (i, 0)),
            pl.BlockSpec((1, _D), lambda i: (0, 0)),
            pl.BlockSpec((1, 1), lambda i: (0, 0)),
        ],
        out_specs=pl.BlockSpec((n,), lambda i: (0,)),
        out_shape=jax.ShapeDtypeStruct((n,), jnp.float32),
    )(z_ins, W, b.reshape(1, 1))


@functools.partial(
    pl.kernel,
    out_type=[
        jax.ShapeDtypeStruct((_BPAD,), jnp.float32),
        jax.ShapeDtypeStruct((_BPAD, _D), jnp.float32),
        jax.ShapeDtypeStruct((_BPAD, _D), jnp.float32),
        jax.ShapeDtypeStruct((_BPAD, _D), jnp.float32),
        jax.ShapeDtypeStruct((_BPAD, _D), jnp.float32),
    ],
    mesh=plsc.VectorSubcoreMesh(core_axis_name="c", subcore_axis_name="s"),
    compiler_params=pltpu.CompilerParams(needs_layout_passes=False),
    scratch_types=[
        pltpu.VMEM((128,), jnp.int32),
        pltpu.VMEM((128,), jnp.int32),
        pltpu.VMEM((_SSTRIDE,), jnp.int32),
        pltpu.VMEM((_WS,), jnp.int32),
        pltpu.VMEM((_WS,), jnp.float32),
        pltpu.VMEM((_BPW,), jnp.float32),
        pltpu.VMEM((_BPW,), jnp.int32),
        pltpu.VMEM((_BPW, _D), jnp.float32),
        pltpu.SemaphoreType.DMA,
    ],
)
def _sc_segment(score_hbm, bag_hbm, t0_hbm, t1_hbm, t2_hbm, t3_hbm,
                m_hbm, o0_hbm, o1_hbm, o2_hbm, o3_hbm,
                sidx_v, samp_v, srch_v, bag_w, score_w, m_loc, loc_loc,
                rows_v, sem):
    wid = lax.axis_index("s") * 2 + lax.axis_index("c")
    blo = wid * _BPW
    t_lo = jnp.minimum(blo, _B)
    t_hi = jnp.minimum(blo + _BPW, _B)

    iota = lax.iota(jnp.int32, 16)
    zeros_i = jnp.zeros((16,), jnp.int32)
    ninf = jnp.full((16,), -jnp.inf, jnp.float32)
    imax = jnp.int32(2**31 - 1)
    bigbag = jnp.int32(2**30)

    for k in range(8):
        lane = iota + 16 * k
        sidx_v[pl.ds(16 * k, 16)] = jnp.where(lane < _NSAMP, lane * _SSTRIDE, 0)
    pltpu.async_copy(bag_hbm.at[sidx_v], samp_v, sem).wait()

    def lower_bound(t):
        def cbody(k, c):
            sv = samp_v[pl.ds(16 * k, 16)]
            lane = iota + 16 * k
            return c + plsc.all_reduce_population_count(
                (lane < _NSAMP) & (sv < t))[0]
        c = lax.fori_loop(0, 8, cbody, jnp.int32(0))
        base = _SSTRIDE * jnp.maximum(c - 1, 0)
        pltpu.sync_copy(bag_hbm.at[pl.ds(pl.multiple_of(base, 8), _SSTRIDE)], srch_v)
        def fbody(k, c2):
            sv = srch_v[pl.ds(16 * k, 16)]
            return c2 + plsc.all_reduce_population_count(sv < t)[0]
        c2 = lax.fori_loop(0, _SSTRIDE // 16, fbody, jnp.int32(0))
        return base + c2

    r0 = lower_bound(t_lo)
    r1 = lower_bound(t_hi)

    for k in range(_BPW // 16):
        loc_loc[pl.ds(16 * k, 16)] = zeros_i

    lane0 = iota == 0

    def emit(j, mv, loc):
        jc = jnp.broadcast_to(jnp.minimum(j, _BPW - 1), (16,))
        plsc.store_scatter(m_loc, [jc], jnp.broadcast_to(mv, (16,)), mask=lane0)
        plsc.store_scatter(loc_loc, [jc], jnp.broadcast_to(loc, (16,)),
                           mask=lane0)

    _gdn = lax.GatherDimensionNumbers(
        offset_dims=(), collapsed_slice_dims=(0,), start_index_map=(0,))

    def splat_lane(vec, lane):
        idx = jnp.broadcast_to(lane, (16,)).astype(jnp.int32)[:, None]
        return lax.gather(vec, idx, _gdn, (1,),
                          mode=lax.GatherScatterMode.PROMISE_IN_BOUNDS)

    @pl.when(r1 > r0)
    def _():
        v0 = r0 - (r0 % 16)
        nv = (r1 - v0 + 15) // 16

        def vbody(k, carry):
            b, s_b, j, m, idx, done = carry
            win_base = pl.multiple_of(jnp.minimum(v0 + 16 * (k - k % _VPW), _N - _WS), 16)

            @pl.when(k % _VPW == 0)
            def _():
                pltpu.sync_copy(bag_hbm.at[pl.ds(win_base, _WS)], bag_w)
                pltpu.sync_copy(score_hbm.at[pl.ds(win_base, _WS)], score_w)

            p = v0 + 16 * k
            rel = p - win_base
            g = bag_w[pl.ds(rel, 16)]
            s = score_w[pl.ds(rel, 16)]
            p_lane = p + iota
            g_eff = jnp.where(p_lane >= r1, bigbag, g)
            off0 = jnp.where(k == 0, r0 - v0, jnp.int32(0))

            b, s_b = lax.cond(
                k == 0,
                lambda b, s_b: (splat_lane(g, off0), r0),
                lambda b, s_b: (b, s_b),
                b, s_b)

            def wcond(st):
                off, _b, _sb, _j, _m, _i, dn = st
                return (off < 16) & (dn == 0)

            def wbody(st):
                off, b, s_b, j, m, idx, done = st
                active = (iota >= off) & (g_eff == b)
                cnt = plsc.all_reduce_population_count(active)[0]
                upd = active & (s > m)
                m2 = jnp.where(upd, s, m)
                idx2 = jnp.where(upd, p_lane, idx)
                off2 = off + cnt

                def fin(off2, b, s_b, j, m2, idx2):
                    mv = jnp.max(m2)
                    cand = jnp.where(m2 == mv, idx2, imax)
                    gm = jnp.min(cand)
                    emit(j, mv, gm - s_b)
                    p_next = p + off2
                    done2 = jnp.where(p_next >= r1, jnp.int32(1), jnp.int32(0))
                    b2 = splat_lane(g, jnp.minimum(off2, 15))
                    return (off2, b2, p_next, j + 1, ninf, zeros_i, done2)

                def cont(off2, b, s_b, j, m2, idx2):
                    return (off2, b, s_b, j, m2, idx2, jnp.int32(0))

                return lax.cond(off2 < 16, fin, cont, off2, b, s_b, j, m2, idx2)

            active0 = (iota >= off0) & (g_eff == b)
            cnt0 = plsc.all_reduce_population_count(active0)[0]

            def fast(b, s_b, j, m, idx, done):
                upd = s > m
                return (b, s_b, j, jnp.where(upd, s, m),
                        jnp.where(upd, p_lane, idx), done)

            def slow(b, s_b, j, m, idx, done):
                st = lax.while_loop(wcond, wbody,
                                    (off0, b, s_b, j, m, idx, done))
                return st[1:]

            return lax.cond(cnt0 == 16, fast, slow, b, s_b, j, m, idx, done)

        init = (zeros_i, jnp.int32(0), jnp.int32(0), ninf, zeros_i,
                jnp.int32(0))
        b, s_b, j, m, idx, done = lax.fori_loop(0, nv, vbody, init)

        @pl.when(done == 0)
        def _():
            mv = jnp.max(m)
            cand = jnp.where(m == mv, idx, imax)
            gm = jnp.min(cand)
            emit(j, mv, gm - s_b)

    pltpu.sync_copy(m_loc, m_hbm.at[pl.ds(pl.multiple_of(blo, 8), _BPW)])
    for tbl, out in ((t0_hbm, o0_hbm), (t1_hbm, o1_hbm),
                     (t2_hbm, o2_hbm), (t3_hbm, o3_hbm)):
        copies = [
            pltpu.async_copy(tbl.at[loc_loc.at[pl.ds(c0, csz)]],
                             rows_v.at[pl.ds(c0, csz)], sem)
            for c0, csz in ((0, 128), (128, 128), (256, 64))]
        for c in copies:
            c.wait()
        pltpu.sync_copy(rows_v, out.at[pl.ds(pl.multiple_of(blo, 8), _BPW)])


def kernel(z_ins, bag_idx, bag_instances, instance_mu, instance_std, W, b):
    score = _matvec(z_ins, W, b)
    m_pad, o_z, o_inst, o_mu, o_std = _sc_segment(
        score, bag_idx, z_ins, bag_instances, instance_mu, instance_std)
    return (m_pad[:_B, None], o_inst[:_B], o_z[:_B], score[:, None],
            o_mu[:_B], o_std[:_B])


# speedup vs baseline: 138.3758x; 1.0815x over previous
import functools

import jax
import jax.numpy as jnp
from jax import lax
from jax.experimental import pallas as pl
from jax.experimental.pallas import tpu as pltpu
from jax.experimental.pallas import tpu_sc as plsc

_N = 320000
_D = 128
_B = 10000
_BN = 12800

_NW = 32
_BPW = 320
_BPAD = _NW * _BPW
_WS = 16384
_VPW = _WS // 16
_SSTRIDE = 2560
_NSAMP = _N // _SSTRIDE


def _matvec_body(x_ref, w_ref, b_ref, o_ref):
    i = pl.program_id(0)
    prod = lax.dot_general(
        w_ref[...].astype(jnp.bfloat16), x_ref[...].astype(jnp.bfloat16),
        (((1,), (1,)), ((), ())), preferred_element_type=jnp.float32)
    o_ref[pl.ds(pl.multiple_of(i * _BN, 128), _BN)] = prod[0] + b_ref[0, 0]


def _matvec(z_ins, W, b):
    n = z_ins.shape[0]
    grid = n // _BN
    return pl.pallas_call(
        _matvec_body,
        grid=(grid,),
        in_specs=[
            pl.BlockSpec((_BN, _D), lambda i: (i, 0)),
            pl.BlockSpec((1, _D), lambda i: (0, 0)),
            pl.BlockSpec((1, 1), lambda i: (0, 0)),
        ],
        out_specs=pl.BlockSpec((n,), lambda i: (0,)),
        out_shape=jax.ShapeDtypeStruct((n,), jnp.float32),
    )(z_ins, W, b.reshape(1, 1))


@functools.partial(
    pl.kernel,
    out_type=[
        jax.ShapeDtypeStruct((_B,), jnp.float32),
        jax.ShapeDtypeStruct((_B, _D), jnp.float32),
        jax.ShapeDtypeStruct((_B, _D), jnp.float32),
        jax.ShapeDtypeStruct((_B, _D), jnp.float32),
        jax.ShapeDtypeStruct((_B, _D), jnp.float32),
    ],
    mesh=plsc.VectorSubcoreMesh(core_axis_name="c", subcore_axis_name="s"),
    compiler_params=pltpu.CompilerParams(needs_layout_passes=False),
    scratch_types=[
        pltpu.VMEM((128,), jnp.int32),
        pltpu.VMEM((128,), jnp.int32),
        pltpu.VMEM((_SSTRIDE,), jnp.int32),
        pltpu.VMEM((_WS,), jnp.int32),
        pltpu.VMEM((_WS,), jnp.float32),
        pltpu.VMEM((_BPW,), jnp.float32),
        pltpu.VMEM((_BPW,), jnp.int32),
        pltpu.VMEM((_BPW, _D), jnp.float32),
        pltpu.SemaphoreType.DMA,
    ],
)
def _sc_segment(score_hbm, bag_hbm, t0_hbm, t1_hbm, t2_hbm, t3_hbm,
                m_hbm, o0_hbm, o1_hbm, o2_hbm, o3_hbm,
                sidx_v, samp_v, srch_v, bag_w, score_w, m_loc, loc_loc,
                rows_v, sem):
    wid = lax.axis_index("s") * 2 + lax.axis_index("c")
    blo = wid * _BPW
    t_lo = jnp.minimum(blo, _B)
    t_hi = jnp.minimum(blo + _BPW, _B)

    iota = lax.iota(jnp.int32, 16)
    zeros_i = jnp.zeros((16,), jnp.int32)
    ninf = jnp.full((16,), -jnp.inf, jnp.float32)
    imax = jnp.int32(2**31 - 1)
    bigbag = jnp.int32(2**30)

    for k in range(8):
        lane = iota + 16 * k
        sidx_v[pl.ds(16 * k, 16)] = jnp.where(lane < _NSAMP, lane * _SSTRIDE, 0)
    pltpu.async_copy(bag_hbm.at[sidx_v], samp_v, sem).wait()

    def lower_bound(t):
        def cbody(k, c):
            sv = samp_v[pl.ds(16 * k, 16)]
            lane = iota + 16 * k
            return c + plsc.all_reduce_population_count(
                (lane < _NSAMP) & (sv < t))[0]
        c = lax.fori_loop(0, 8, cbody, jnp.int32(0))
        base = _SSTRIDE * jnp.maximum(c - 1, 0)
        pltpu.sync_copy(bag_hbm.at[pl.ds(pl.multiple_of(base, 8), _SSTRIDE)], srch_v)
        def fbody(k, c2):
            sv = srch_v[pl.ds(16 * k, 16)]
            return c2 + plsc.all_reduce_population_count(sv < t)[0]
        c2 = lax.fori_loop(0, _SSTRIDE // 16, fbody, jnp.int32(0))
        return base + c2

    r0 = lower_bound(t_lo)
    r1 = lower_bound(t_hi)

    for k in range(_BPW // 16):
        loc_loc[pl.ds(16 * k, 16)] = zeros_i

    lane0 = iota == 0

    def emit(j, mv, loc):
        jc = jnp.broadcast_to(jnp.minimum(j, _BPW - 1), (16,))
        plsc.store_scatter(m_loc, [jc], jnp.broadcast_to(mv, (16,)), mask=lane0)
        plsc.store_scatter(loc_loc, [jc], jnp.broadcast_to(loc, (16,)),
                           mask=lane0)

    _gdn = lax.GatherDimensionNumbers(
        offset_dims=(), collapsed_slice_dims=(0,), start_index_map=(0,))

    def splat_lane(vec, lane):
        idx = jnp.broadcast_to(lane, (16,)).astype(jnp.int32)[:, None]
        return lax.gather(vec, idx, _gdn, (1,),
                          mode=lax.GatherScatterMode.PROMISE_IN_BOUNDS)

    @pl.when(r1 > r0)
    def _():
        v0 = r0 - (r0 % 16)
        nv = (r1 - v0 + 15) // 16

        def vbody(k, carry):
            b, s_b, j, m, idx, done = carry
            win_base = pl.multiple_of(jnp.minimum(v0 + 16 * (k - k % _VPW), _N - _WS), 16)

            @pl.when(k % _VPW == 0)
            def _():
                pltpu.sync_copy(bag_hbm.at[pl.ds(win_base, _WS)], bag_w)
                pltpu.sync_copy(score_hbm.at[pl.ds(win_base, _WS)], score_w)

            p = v0 + 16 * k
            rel = p - win_base
            g = bag_w[pl.ds(rel, 16)]
            s = score_w[pl.ds(rel, 16)]
            p_lane = p + iota
            g_eff = jnp.where(p_lane >= r1, bigbag, g)
            off0 = jnp.where(k == 0, r0 - v0, jnp.int32(0))

            b, s_b = lax.cond(
                k == 0,
                lambda b, s_b: (splat_lane(g, off0), r0),
                lambda b, s_b: (b, s_b),
                b, s_b)

            def wcond(st):
                off, _b, _sb, _j, _m, _i, dn = st
                return (off < 16) & (dn == 0)

            def wbody(st):
                off, b, s_b, j, m, idx, done = st
                active = (iota >= off) & (g_eff == b)
                cnt = plsc.all_reduce_population_count(active)[0]
                upd = active & (s > m)
                m2 = jnp.where(upd, s, m)
                idx2 = jnp.where(upd, p_lane, idx)
                off2 = off + cnt

                def fin(off2, b, s_b, j, m2, idx2):
                    mv = jnp.max(m2)
                    cand = jnp.where(m2 == mv, idx2, imax)
                    gm = jnp.min(cand)
                    emit(j, mv, gm - s_b)
                    p_next = p + off2
                    done2 = jnp.where(p_next >= r1, jnp.int32(1), jnp.int32(0))
                    b2 = splat_lane(g, jnp.minimum(off2, 15))
                    return (off2, b2, p_next, j + 1, ninf, zeros_i, done2)

                def cont(off2, b, s_b, j, m2, idx2):
                    return (off2, b, s_b, j, m2, idx2, jnp.int32(0))

                return lax.cond(off2 < 16, fin, cont, off2, b, s_b, j, m2, idx2)

            active0 = (iota >= off0) & (g_eff == b)
            cnt0 = plsc.all_reduce_population_count(active0)[0]

            def fast(b, s_b, j, m, idx, done):
                upd = s > m
                return (b, s_b, j, jnp.where(upd, s, m),
                        jnp.where(upd, p_lane, idx), done)

            def slow(b, s_b, j, m, idx, done):
                st = lax.while_loop(wcond, wbody,
                                    (off0, b, s_b, j, m, idx, done))
                return st[1:]

            return lax.cond(cnt0 == 16, fast, slow, b, s_b, j, m, idx, done)

        init = (zeros_i, jnp.int32(0), jnp.int32(0), ninf, zeros_i,
                jnp.int32(0))
        b, s_b, j, m, idx, done = lax.fori_loop(0, nv, vbody, init)

        @pl.when(done == 0)
        def _():
            mv = jnp.max(m)
            cand = jnp.where(m == mv, idx, imax)
            gm = jnp.min(cand)
            emit(j, mv, gm - s_b)

    tails = _B - 31 * _BPW

    @pl.when(wid < _NW - 1)
    def _():
        pltpu.sync_copy(m_loc, m_hbm.at[pl.ds(pl.multiple_of(blo, 8), _BPW)])
        for tbl, out in ((t0_hbm, o0_hbm), (t1_hbm, o1_hbm),
                         (t2_hbm, o2_hbm), (t3_hbm, o3_hbm)):
            copies = [
                pltpu.async_copy(tbl.at[loc_loc.at[pl.ds(c0, csz)]],
                                 rows_v.at[pl.ds(c0, csz)], sem)
                for c0, csz in ((0, 128), (128, 128), (256, 64))]
            for c in copies:
                c.wait()
            pltpu.sync_copy(rows_v,
                            out.at[pl.ds(pl.multiple_of(blo, 8), _BPW)])

    @pl.when(wid == _NW - 1)
    def _():
        base = jnp.int32(_B - tails)
        pltpu.sync_copy(m_loc.at[pl.ds(0, tails)],
                        m_hbm.at[pl.ds(pl.multiple_of(base, 8), tails)])
        for tbl, out in ((t0_hbm, o0_hbm), (t1_hbm, o1_hbm),
                         (t2_hbm, o2_hbm), (t3_hbm, o3_hbm)):
            pltpu.async_copy(tbl.at[loc_loc.at[pl.ds(0, tails)]],
                             rows_v.at[pl.ds(0, tails)], sem).wait()
            pltpu.sync_copy(rows_v.at[pl.ds(0, tails)],
                            out.at[pl.ds(pl.multiple_of(base, 8), tails)])


def kernel(z_ins, bag_idx, bag_instances, instance_mu, instance_std, W, b):
    score = _matvec(z_ins, W, b)
    m_out, o_z, o_inst, o_mu, o_std = _sc_segment(
        score, bag_idx, z_ins, bag_instances, instance_mu, instance_std)
    return (m_out[:, None], o_inst, o_z, score[:, None], o_mu, o_std)
